# baseline (speedup 1.0000x reference)
import functools

import jax
import jax.numpy as jnp
from jax import lax
from jax.experimental import pallas as pl
from jax.experimental.pallas import tpu as pltpu
from jax.experimental.pallas import tpu_sc as plsc

B = 4096
L_ITEM = 20
L_CTX = 50
L_CTXP = 64
EMBED = 32
CTXD = 32

NC = 2
NS = 16
NW = NC * NS

BPW = B // NW
ITEM_PW = BPW * L_ITEM
CTX_PW = BPW * L_CTXP
CH = 128
ITEM_CHUNKS = ITEM_PW // CH
CTX_CHUNKS = CTX_PW // CH
RPC = CH // L_CTXP
CNB = 16
SDEPTH = 5
GDEPTH = CNB - SDEPTH
CHS = 64
ICH_SUP = ITEM_PW // CHS
IBS = 8
IGDS = 6
PACK = 4


@functools.cache
def _sc_ctx_fn():
    mesh = plsc.VectorSubcoreMesh(core_axis_name="c", subcore_axis_name="s")

    @functools.partial(
        pl.kernel,
        mesh=mesh,
        out_type=jax.ShapeDtypeStruct((B, CTXD), jnp.float32),
        scratch_types=[
            pltpu.VMEM((CTX_PW,), jnp.int32),
            pltpu.VMEM((CNB, CH, CTXD), jnp.float32),
            pltpu.VMEM((CNB, CH), jnp.int32),
            pltpu.VMEM_SHARED((NS * BPW, CTXD), jnp.float32),
            pltpu.SemaphoreType.DMA,
            pltpu.SemaphoreType.DMA,
        ],
        compiler_params=pltpu.CompilerParams(use_tc_tiling_on_sc=False),
    )
    def _sc_ctx(ctx_idx, set_tab, sum_out,
                cidx_v, ctx_bufs, sidx_v, acc_sh, csem, ssem):
        cid = lax.axis_index("c")
        sid = lax.axis_index("s")
        wid = sid * NC + cid

        with jax.named_scope("adjust"):
            pltpu.sync_copy(ctx_idx.at[pl.ds(wid * CTX_PW, CTX_PW)], cidx_v)

            def adjust_body(g, carry):
                v = cidx_v[pl.ds(g * 16, 16)]
                cidx_v[pl.ds(g * 16, 16)] = jnp.maximum(v - 1, 0)
                return carry

            lax.fori_loop(0, CTX_PW // 16, adjust_body, 0)

        with jax.named_scope("zero_acc"):
            def zero_body(r, carry):
                ctx_bufs[0, r, pl.ds(0, 16)] = jnp.zeros((16,), jnp.float32)
                ctx_bufs[0, r, pl.ds(16, 16)] = jnp.zeros((16,), jnp.float32)
                return carry

            lax.fori_loop(0, CH, zero_body, 0)
            pltpu.sync_copy(ctx_bufs.at[0],
                            acc_sh.at[pl.ds(sid * BPW, BPW)])

        def fire_ctx(c, buf):
            pltpu.async_copy(
                set_tab.at[cidx_v.at[pl.ds(c * CH, CH)]],
                ctx_bufs.at[buf], csem)

        with jax.named_scope("ctx_prime"):
            for b in range(GDEPTH):
                fire_ctx(b, b)

        with jax.named_scope("ctx_loop"):
            def ctx_group(g, carry):
                for b in range(CNB):
                    c = g * CNB + b
                    pltpu.make_async_copy(set_tab.at[pl.ds(0, CH)],
                                          ctx_bufs.at[b], csem).wait()
                    base = sid * BPW + RPC * c
                    for t in range(CH // 16):
                        sidx_v[b, pl.ds(t * 16, 16)] = (
                            jnp.zeros((16,), jnp.int32)
                            + (base + (1 if t >= CH // 32 else 0)))
                    pltpu.async_copy(ctx_bufs.at[b], acc_sh.at[sidx_v.at[b]],
                                     ssem, add=True)

                    @pl.when(c >= SDEPTH)
                    def _():
                        pltpu.make_async_copy(
                            ctx_bufs.at[b], acc_sh.at[pl.ds(0, CH)],
                            ssem).wait()

                    @pl.when(c + GDEPTH < CTX_CHUNKS)
                    def _():
                        fire_ctx(c + GDEPTH, (b + GDEPTH) % CNB)
                return carry

            lax.fori_loop(0, CTX_CHUNKS // CNB, ctx_group, 0)

        with jax.named_scope("tail"):
            for _ in range(SDEPTH):
                pltpu.make_async_copy(ctx_bufs.at[0], acc_sh.at[pl.ds(0, CH)],
                                      ssem).wait()
            pltpu.sync_copy(acc_sh.at[pl.ds(sid * BPW, BPW)],
                            sum_out.at[pl.ds(wid * BPW, BPW)])

    return _sc_ctx


@functools.cache
def _sc_item_fn():
    mesh = plsc.VectorSubcoreMesh(core_axis_name="c", subcore_axis_name="s")

    @functools.partial(
        pl.kernel,
        mesh=mesh,
        out_type=jax.ShapeDtypeStruct((B * L_ITEM, 128), jnp.float32),
        scratch_types=[
            pltpu.VMEM((ITEM_PW,), jnp.int32),
            pltpu.VMEM((IBS, CHS, 128), jnp.float32),
            pltpu.SemaphoreType.DMA,
            pltpu.SemaphoreType.DMA,
        ],
        compiler_params=pltpu.CompilerParams(use_tc_tiling_on_sc=False),
    )
    def _sc_item(item_idx, item_sup, item_out, iidx_v, item_bufs, isem, wsem):
        cid = lax.axis_index("c")
        sid = lax.axis_index("s")
        wid = sid * NC + cid

        with jax.named_scope("item_fire"):
            pltpu.sync_copy(item_idx.at[pl.ds(wid * ITEM_PW, ITEM_PW)], iidx_v)

            def iadj(g, carry):
                iidx_v[pl.ds(g * 16, 16)] = iidx_v[pl.ds(g * 16, 16)] >> 2
                return carry

            lax.fori_loop(0, ITEM_PW // 16, iadj, 0)

            def fire_item(j, buf):
                pltpu.async_copy(
                    item_sup.at[iidx_v.at[pl.ds(j * CHS, CHS)]],
                    item_bufs.at[buf], isem)

            for j in range(IGDS):
                fire_item(j, j)

        with jax.named_scope("item_drain"):
            for j in range(ICH_SUP):
                pltpu.make_async_copy(item_sup.at[pl.ds(0, CHS)],
                                      item_bufs.at[j % IBS], isem).wait()
                pltpu.async_copy(
                    item_bufs.at[j % IBS],
                    item_out.at[pl.ds(wid * ITEM_PW + j * CHS, CHS)], wsem)
                g = j + IGDS
                if g < ICH_SUP:
                    if g >= IBS:
                        pltpu.make_async_copy(
                            item_bufs.at[0],
                            item_out.at[pl.ds(0, CHS)], wsem).wait()
                    fire_item(g, g % IBS)
            for _ in range(IBS):
                pltpu.make_async_copy(
                    item_bufs.at[0],
                    item_out.at[pl.ds(0, CHS)], wsem).wait()

    return _sc_item


BB = 256


def _tc_body(idx_ref, iidx_ref, it_ref, sum_ref, tab_ref, w1, b1, w2, b2,
             w3, b3, out_ref):
    maskf = (idx_ref[...] > 0).astype(jnp.float32)
    nz = jnp.sum(maskf, axis=-1, keepdims=True)
    tab = tab_ref[...]
    padsum = jnp.sum(tab[L_CTX - 1:L_CTXP - 1, :], axis=0, keepdims=True)
    summed = (sum_ref[...] - padsum
              - (float(L_CTX) - nz) * tab[0:1, :])
    sq = jnp.sum(summed * summed, axis=-1, keepdims=True)
    normalized = summed * lax.rsqrt(jnp.maximum(sq, 1e-4))
    h = jnp.maximum(
        jnp.dot(normalized, w1[...], preferred_element_type=jnp.float32) + b1[...], 0.0)
    h = jnp.maximum(
        jnp.dot(h, w2[...], preferred_element_type=jnp.float32) + b2[...], 0.0)
    ce = jnp.dot(h, w3[...], preferred_element_type=jnp.float32) + b3[...]
    it128 = it_ref[...]
    sel = (iidx_ref[...] & 3)[:, :, None]
    it = jnp.zeros((BB, L_ITEM, EMBED), jnp.float32)
    for a in range(4):
        it = it + jnp.where(sel == a,
                            it128[:, :, a * EMBED:(a + 1) * EMBED], 0.0)
    diff = it - ce[:, None, :]
    d = jnp.sqrt(jnp.sum(diff * diff, axis=-1))
    out_ref[...] = 1.0 - jnp.tanh(d)


def _tc_compute(ctx_idx, item_idx, item_rows, summed, tab64,
                W1, b1, W2, b2, W3, b3):
    grid = (B // BB,)
    return pl.pallas_call(
        _tc_body,
        grid=grid,
        in_specs=[
            pl.BlockSpec((BB, L_CTX), lambda i: (i, 0)),
            pl.BlockSpec((BB, L_ITEM), lambda i: (i, 0)),
            pl.BlockSpec((BB, L_ITEM, 128), lambda i: (i, 0, 0)),
            pl.BlockSpec((BB, CTXD), lambda i: (i, 0)),
            pl.BlockSpec((L_CTXP, CTXD), lambda i: (0, 0)),
            pl.BlockSpec((CTXD, 2 * CTXD), lambda i: (0, 0)),
            pl.BlockSpec((1, 2 * CTXD), lambda i: (0, 0)),
            pl.BlockSpec((2 * CTXD, 4 * CTXD), lambda i: (0, 0)),
            pl.BlockSpec((1, 4 * CTXD), lambda i: (0, 0)),
            pl.BlockSpec((4 * CTXD, EMBED), lambda i: (0, 0)),
            pl.BlockSpec((1, EMBED), lambda i: (0, 0)),
        ],
        out_specs=pl.BlockSpec((BB, L_ITEM), lambda i: (i, 0)),
        out_shape=jax.ShapeDtypeStruct((B, L_ITEM), jnp.float32),
    )(ctx_idx, item_idx, item_rows, summed, tab64,
      W1, b1, W2, b2, W3, b3)


def kernel(item_indices, context_indices, item_table, set_table,
           W1, b1, W2, b2, W3, b3):
    pad_block = jnp.broadcast_to(
        jnp.arange(L_CTX, L_CTXP, dtype=jnp.int32), (B, L_CTXP - L_CTX))
    ctx_pad = jnp.concatenate([context_indices, pad_block], axis=1)
    summed = _sc_ctx_fn()(ctx_pad.reshape(-1), set_table)
    item_sup = item_table.reshape(1000000 // PACK, 128)
    item_rows = _sc_item_fn()(item_indices.reshape(-1), item_sup)
    return _tc_compute(
        context_indices, item_indices,
        item_rows.reshape(B, L_ITEM, 128),
        summed, set_table[:L_CTXP],
        W1, b1.reshape(1, -1), W2, b2.reshape(1, -1), W3, b3.reshape(1, -1))


# speedup vs baseline: 1.0779x; 1.0779x over previous
import functools

import jax
import jax.numpy as jnp
from jax import lax
from jax.experimental import pallas as pl
from jax.experimental.pallas import tpu as pltpu
from jax.experimental.pallas import tpu_sc as plsc

B = 4096
L_ITEM = 20
L_CTX = 50
L_CTXP = 64
EMBED = 32
CTXD = 32

NC = 2
NS = 16
NW = NC * NS

BPW = B // NW
ITEM_PW = BPW * L_ITEM
CTX_PW = BPW * L_CTXP
CH = 128
ITEM_CHUNKS = ITEM_PW // CH
CTX_CHUNKS = CTX_PW // CH
RPC = CH // L_CTXP
CNB = 16
SDEPTH = 5
GDEPTH = CNB - SDEPTH
IBUF = 8
IWD = 2
IGD = IBUF - IWD


@functools.cache
def _sc_ctx_fn():
    mesh = plsc.VectorSubcoreMesh(core_axis_name="c", subcore_axis_name="s")

    @functools.partial(
        pl.kernel,
        mesh=mesh,
        out_type=jax.ShapeDtypeStruct((B, CTXD), jnp.float32),
        scratch_types=[
            pltpu.VMEM((CTX_PW,), jnp.int32),
            pltpu.VMEM((CNB, CH, CTXD), jnp.float32),
            pltpu.VMEM((CNB, CH), jnp.int32),
            pltpu.VMEM_SHARED((NS * BPW, CTXD), jnp.float32),
            pltpu.SemaphoreType.DMA,
            pltpu.SemaphoreType.DMA,
        ],
        compiler_params=pltpu.CompilerParams(use_tc_tiling_on_sc=False),
    )
    def _sc_ctx(ctx_idx, set_tab, sum_out,
                cidx_v, ctx_bufs, sidx_v, acc_sh, csem, ssem):
        cid = lax.axis_index("c")
        sid = lax.axis_index("s")
        wid = sid * NC + cid

        with jax.named_scope("adjust"):
            pltpu.sync_copy(ctx_idx.at[pl.ds(wid * CTX_PW, CTX_PW)], cidx_v)

            def adjust_body(g, carry):
                v = cidx_v[pl.ds(g * 16, 16)]
                cidx_v[pl.ds(g * 16, 16)] = jnp.maximum(v - 1, 0)
                return carry

            lax.fori_loop(0, CTX_PW // 16, adjust_body, 0)

        with jax.named_scope("zero_acc"):
            def zero_body(r, carry):
                ctx_bufs[0, r, pl.ds(0, 16)] = jnp.zeros((16,), jnp.float32)
                ctx_bufs[0, r, pl.ds(16, 16)] = jnp.zeros((16,), jnp.float32)
                return carry

            lax.fori_loop(0, CH, zero_body, 0)
            pltpu.sync_copy(ctx_bufs.at[0],
                            acc_sh.at[pl.ds(sid * BPW, BPW)])

        def fire_ctx(c, buf):
            pltpu.async_copy(
                set_tab.at[cidx_v.at[pl.ds(c * CH, CH)]],
                ctx_bufs.at[buf], csem)

        with jax.named_scope("ctx_prime"):
            for b in range(GDEPTH):
                fire_ctx(b, b)

        with jax.named_scope("ctx_loop"):
            def ctx_group(g, carry):
                for b in range(CNB):
                    c = g * CNB + b
                    pltpu.make_async_copy(set_tab.at[pl.ds(0, CH)],
                                          ctx_bufs.at[b], csem).wait()
                    base = sid * BPW + RPC * c
                    for t in range(CH // 16):
                        sidx_v[b, pl.ds(t * 16, 16)] = (
                            jnp.zeros((16,), jnp.int32)
                            + (base + (1 if t >= CH // 32 else 0)))
                    pltpu.async_copy(ctx_bufs.at[b], acc_sh.at[sidx_v.at[b]],
                                     ssem, add=True)

                    @pl.when(c >= SDEPTH)
                    def _():
                        pltpu.make_async_copy(
                            ctx_bufs.at[b], acc_sh.at[pl.ds(0, CH)],
                            ssem).wait()

                    @pl.when(c + GDEPTH < CTX_CHUNKS)
                    def _():
                        fire_ctx(c + GDEPTH, (b + GDEPTH) % CNB)
                return carry

            lax.fori_loop(0, CTX_CHUNKS // CNB, ctx_group, 0)

        with jax.named_scope("tail"):
            for _ in range(SDEPTH):
                pltpu.make_async_copy(ctx_bufs.at[0], acc_sh.at[pl.ds(0, CH)],
                                      ssem).wait()
            pltpu.sync_copy(acc_sh.at[pl.ds(sid * BPW, BPW)],
                            sum_out.at[pl.ds(wid * BPW, BPW)])

    return _sc_ctx


@functools.cache
def _sc_item_fn():
    mesh = plsc.VectorSubcoreMesh(core_axis_name="c", subcore_axis_name="s")

    @functools.partial(
        pl.kernel,
        mesh=mesh,
        out_type=jax.ShapeDtypeStruct((B * L_ITEM, EMBED), jnp.float32),
        scratch_types=[
            pltpu.VMEM((ITEM_PW,), jnp.int32),
            pltpu.VMEM((ITEM_CHUNKS, CH, EMBED), jnp.float32),
            pltpu.SemaphoreType.DMA,
            pltpu.SemaphoreType.DMA,
        ],
        compiler_params=pltpu.CompilerParams(use_tc_tiling_on_sc=False),
    )
    def _sc_item(item_idx, item_tab, item_out, iidx_v, item_bufs, isem, wsem):
        cid = lax.axis_index("c")
        sid = lax.axis_index("s")
        wid = sid * NC + cid

        with jax.named_scope("item_fire"):
            pltpu.sync_copy(item_idx.at[pl.ds(wid * ITEM_PW, ITEM_PW)], iidx_v)
            for j in range(ITEM_CHUNKS):
                pltpu.async_copy(
                    item_tab.at[iidx_v.at[pl.ds(j * CH, CH)]],
                    item_bufs.at[j], isem)

        with jax.named_scope("item_drain"):
            for j in range(ITEM_CHUNKS):
                pltpu.make_async_copy(item_tab.at[pl.ds(0, CH)],
                                      item_bufs.at[j], isem).wait()
                pltpu.async_copy(
                    item_bufs.at[j],
                    item_out.at[pl.ds(wid * ITEM_PW + j * CH, CH)], wsem)
            for j in range(ITEM_CHUNKS):
                pltpu.make_async_copy(
                    item_bufs.at[j],
                    item_out.at[pl.ds(wid * ITEM_PW + j * CH, CH)],
                    wsem).wait()

    return _sc_item


BB = 256


def _tc_body(idx_ref, it_ref, sum_ref, tab_ref, w1, b1, w2, b2, w3, b3,
             out_ref):
    maskf = (idx_ref[...] > 0).astype(jnp.float32)
    nz = jnp.sum(maskf, axis=-1, keepdims=True)
    tab = tab_ref[...]
    padsum = jnp.sum(tab[L_CTX - 1:L_CTXP - 1, :], axis=0, keepdims=True)
    summed = (sum_ref[...] - padsum
              - (float(L_CTX) - nz) * tab[0:1, :])
    sq = jnp.sum(summed * summed, axis=-1, keepdims=True)
    normalized = summed * lax.rsqrt(jnp.maximum(sq, 1e-4))
    h = jnp.maximum(
        jnp.dot(normalized, w1[...], preferred_element_type=jnp.float32) + b1[...], 0.0)
    h = jnp.maximum(
        jnp.dot(h, w2[...], preferred_element_type=jnp.float32) + b2[...], 0.0)
    ce = jnp.dot(h, w3[...], preferred_element_type=jnp.float32) + b3[...]
    it = it_ref[...]
    diff = it - ce[:, None, :]
    d = jnp.sqrt(jnp.sum(diff * diff, axis=-1))
    out_ref[...] = 1.0 - jnp.tanh(d)


def _tc_compute(ctx_idx, item_rows, summed, tab64, W1, b1, W2, b2, W3, b3):
    grid = (B // BB,)
    return pl.pallas_call(
        _tc_body,
        grid=grid,
        in_specs=[
            pl.BlockSpec((BB, L_CTX), lambda i: (i, 0)),
            pl.BlockSpec((BB, L_ITEM, EMBED), lambda i: (i, 0, 0)),
            pl.BlockSpec((BB, CTXD), lambda i: (i, 0)),
            pl.BlockSpec((L_CTXP, CTXD), lambda i: (0, 0)),
            pl.BlockSpec((CTXD, 2 * CTXD), lambda i: (0, 0)),
            pl.BlockSpec((1, 2 * CTXD), lambda i: (0, 0)),
            pl.BlockSpec((2 * CTXD, 4 * CTXD), lambda i: (0, 0)),
            pl.BlockSpec((1, 4 * CTXD), lambda i: (0, 0)),
            pl.BlockSpec((4 * CTXD, EMBED), lambda i: (0, 0)),
            pl.BlockSpec((1, EMBED), lambda i: (0, 0)),
        ],
        out_specs=pl.BlockSpec((BB, L_ITEM), lambda i: (i, 0)),
        out_shape=jax.ShapeDtypeStruct((B, L_ITEM), jnp.float32),
    )(ctx_idx, item_rows, summed, tab64, W1, b1, W2, b2, W3, b3)


def kernel(item_indices, context_indices, item_table, set_table,
           W1, b1, W2, b2, W3, b3):
    pad_block = jnp.broadcast_to(
        jnp.arange(L_CTX, L_CTXP, dtype=jnp.int32), (B, L_CTXP - L_CTX))
    ctx_pad = jnp.concatenate([context_indices, pad_block], axis=1)
    summed = _sc_ctx_fn()(ctx_pad.reshape(-1), set_table)
    item_rows = _sc_item_fn()(item_indices.reshape(-1), item_table)
    return _tc_compute(
        context_indices,
        item_rows.reshape(B, L_ITEM, EMBED),
        summed, set_table[:L_CTXP],
        W1, b1.reshape(1, -1), W2, b2.reshape(1, -1), W3, b3.reshape(1, -1))
